# Initial kernel scaffold; baseline (speedup 1.0000x reference)
import jax
import jax.numpy as jnp
from jax.experimental import pallas as pl


def kernel(pos, feat, member_idx, cluster_mask, learned_prob, stride, pe_idx, reserve_num, pre_table, W1, b1, ln1_g, ln1_b, norm_g, norm_b, W2, b2):
    raise NotImplementedError("write your pallas kernel here")


# speedup vs baseline: 1.0153x; 1.0153x over previous
import functools

import jax
import jax.numpy as jnp
from jax.experimental import pallas as pl
from jax.experimental.pallas import tpu as pltpu

DS_RATE = 0.25


def _tail_kernel(x_ref, g_ref, b_ref, w_ref, b2_ref, o_ref):
    x = x_ref[...]
    m = jnp.mean(x, axis=-1, keepdims=True)
    v = jnp.mean((x - m) ** 2, axis=-1, keepdims=True)
    xn = (x - m) * jax.lax.rsqrt(v + 1e-5) * g_ref[...] + b_ref[...]
    o_ref[...] = jnp.dot(xn, w_ref[...], preferred_element_type=jnp.float32) + b2_ref[...]


def _ln_proj(x, norm_g, norm_b, W2, b2):
    rows, d = x.shape
    out_d = W2.shape[1]
    blk = 1024
    return pl.pallas_call(
        _tail_kernel,
        grid=(rows // blk,),
        in_specs=[
            pl.BlockSpec((blk, d), lambda i: (i, 0)),
            pl.BlockSpec((1, d), lambda i: (0, 0)),
            pl.BlockSpec((1, d), lambda i: (0, 0)),
            pl.BlockSpec((d, out_d), lambda i: (0, 0)),
            pl.BlockSpec((1, out_d), lambda i: (0, 0)),
        ],
        out_specs=pl.BlockSpec((blk, out_d), lambda i: (i, 0)),
        out_shape=jax.ShapeDtypeStruct((rows, out_d), jnp.float32),
    )(x, norm_g.reshape(1, d), norm_b.reshape(1, d), W2, b2.reshape(1, out_d))


def kernel(pos, feat, member_idx, cluster_mask, learned_prob, stride, pe_idx,
           reserve_num, pre_table, W1, b1, ln1_g, ln1_b, norm_g, norm_b, W2, b2):
    b, n, c = feat.shape
    d = pos.shape[2]
    nbhd = member_idx.shape[-1]
    keep_num = int(n * DS_RATE)
    final_prob = learned_prob.reshape(b, n)
    _, sample_idx = jax.lax.top_k(final_prob, keep_num)
    idx = sample_idx[:, :, None]
    pos_down = jnp.take_along_axis(pos, jnp.broadcast_to(idx, (b, keep_num, d)), axis=1)
    member_idx_d = jnp.take_along_axis(member_idx, jnp.broadcast_to(idx, (b, keep_num, nbhd)), axis=1)
    pe_idx_d = jnp.take_along_axis(pe_idx, jnp.broadcast_to(idx, (b, keep_num, nbhd)), axis=1)
    cluster_mask_d = jnp.take_along_axis(cluster_mask, jnp.broadcast_to(idx, (b, keep_num, nbhd)), axis=1)
    lp = jnp.take_along_axis(learned_prob, member_idx_d.reshape(b, -1, 1), axis=1).reshape(b, keep_num, nbhd, 1)

    def layernorm(x, g, bb, eps=1e-5):
        m = jnp.mean(x, axis=-1, keepdims=True)
        v = jnp.var(x, axis=-1, keepdims=True)
        return (x - m) / jnp.sqrt(v + eps) * g + bb

    wt = pre_table @ W1 + b1
    wt = layernorm(wt, ln1_g, ln1_b)
    wt = jax.nn.gelu(wt, approximate=False)
    weights = jnp.take(wt, pe_idx_d, axis=0)
    lp = lp * cluster_mask_d[..., None]
    weights = weights * lp
    gathered = jnp.take_along_axis(feat, member_idx_d.reshape(b, -1)[:, :, None], axis=1).reshape(b, keep_num, nbhd, c)
    feat_down = jnp.einsum('bnki,bnkc->bnic', weights, gathered).reshape(b, keep_num, -1)
    feat_down = _ln_proj(feat_down.reshape(b * keep_num, -1), norm_g, norm_b, W2, b2).reshape(b, keep_num, -1)
    return pos_down, feat_down


# speedup vs baseline: 2.5139x; 2.4759x over previous
import functools

import jax
import jax.numpy as jnp
from jax import lax
from jax.experimental import pallas as pl
from jax.experimental.pallas import tpu as pltpu
from jax.experimental.pallas import tpu_sc as plsc

DS_RATE = 0.25

_SC_INFO = plsc.get_sparse_core_info()
_NC, _NS = _SC_INFO.num_cores, _SC_INFO.num_subcores
_NW = _NC * _NS


def _sc_gather_rows(table, gidx):
    V, D = table.shape
    R = gidx.shape[0]
    CH = 128
    rows_per_w = R // _NW
    n_chunks = rows_per_w // CH
    assert rows_per_w % CH == 0 and R % _NW == 0

    mesh = plsc.VectorSubcoreMesh(core_axis_name="c", subcore_axis_name="s")

    @functools.partial(
        pl.kernel,
        out_type=jax.ShapeDtypeStruct((R, D), jnp.float32),
        mesh=mesh,
        scratch_types=[
            pltpu.VMEM((rows_per_w,), jnp.int32),
            pltpu.VMEM((CH, D), jnp.float32),
            pltpu.VMEM((CH, D), jnp.float32),
            pltpu.SemaphoreType.DMA,
            pltpu.SemaphoreType.DMA,
            pltpu.SemaphoreType.DMA,
            pltpu.SemaphoreType.DMA,
        ],
    )
    def k(table_hbm, gidx_hbm, out_hbm, idx_v, buf0, buf1, g0, g1, o0, o1):
        wid = lax.axis_index("s") * _NC + lax.axis_index("c")
        base = wid * rows_per_w
        pltpu.sync_copy(gidx_hbm.at[pl.ds(base, rows_per_w)], idx_v)
        bufs = (buf0, buf1)
        gsems = (g0, g1)
        osems = (o0, o1)

        def start_gather(c, p):
            pltpu.async_copy(table_hbm.at[idx_v.at[pl.ds(c * CH, CH)]],
                             bufs[p], gsems[p])

        def gather_wait(p):
            pltpu.make_async_copy(table_hbm.at[idx_v.at[pl.ds(0, CH)]],
                                  bufs[p], gsems[p]).wait()

        def start_out(c, p):
            pltpu.async_copy(bufs[p], out_hbm.at[pl.ds(base + c * CH, CH)],
                             osems[p])

        def out_wait(p):
            pltpu.make_async_copy(bufs[p],
                                  out_hbm.at[pl.ds(base, CH)], osems[p]).wait()

        start_gather(0, 0)

        def iter2(it, carry):
            for sub in range(2):
                c = 2 * it + sub
                p = sub
                q = 1 - sub
                @pl.when(c >= 1)
                def _():
                    out_wait(q)

                @pl.when(c + 1 < n_chunks)
                def _():
                    start_gather(c + 1, q)

                gather_wait(p)
                start_out(c, p)
            return carry

        lax.fori_loop(0, n_chunks // 2, iter2, 0)
        out_wait((n_chunks - 1) % 2)

    return k(table, gidx)


def _tail_kernel(x_ref, g_ref, b_ref, w_ref, b2_ref, o_ref):
    x = x_ref[...]
    m = jnp.mean(x, axis=-1, keepdims=True)
    v = jnp.mean((x - m) ** 2, axis=-1, keepdims=True)
    xn = (x - m) * jax.lax.rsqrt(v + 1e-5) * g_ref[...] + b_ref[...]
    o_ref[...] = jnp.dot(xn, w_ref[...], preferred_element_type=jnp.float32) + b2_ref[...]


def _ln_proj(x, norm_g, norm_b, W2, b2):
    rows, d = x.shape
    out_d = W2.shape[1]
    blk = 1024
    return pl.pallas_call(
        _tail_kernel,
        grid=(rows // blk,),
        in_specs=[
            pl.BlockSpec((blk, d), lambda i: (i, 0)),
            pl.BlockSpec((1, d), lambda i: (0, 0)),
            pl.BlockSpec((1, d), lambda i: (0, 0)),
            pl.BlockSpec((d, out_d), lambda i: (0, 0)),
            pl.BlockSpec((1, out_d), lambda i: (0, 0)),
        ],
        out_specs=pl.BlockSpec((blk, out_d), lambda i: (i, 0)),
        out_shape=jax.ShapeDtypeStruct((rows, out_d), jnp.float32),
    )(x, norm_g.reshape(1, d), norm_b.reshape(1, d), W2, b2.reshape(1, out_d))


def kernel(pos, feat, member_idx, cluster_mask, learned_prob, stride, pe_idx,
           reserve_num, pre_table, W1, b1, ln1_g, ln1_b, norm_g, norm_b, W2, b2):
    b, n, c = feat.shape
    d = pos.shape[2]
    nbhd = member_idx.shape[-1]
    keep_num = int(n * DS_RATE)
    final_prob = learned_prob.reshape(b, n)
    _, sample_idx = jax.lax.top_k(final_prob, keep_num)
    idx = sample_idx[:, :, None]
    pos_down = jnp.take_along_axis(pos, jnp.broadcast_to(idx, (b, keep_num, d)), axis=1)
    member_idx_d = jnp.take_along_axis(member_idx, jnp.broadcast_to(idx, (b, keep_num, nbhd)), axis=1)
    pe_idx_d = jnp.take_along_axis(pe_idx, jnp.broadcast_to(idx, (b, keep_num, nbhd)), axis=1)
    cluster_mask_d = jnp.take_along_axis(cluster_mask, jnp.broadcast_to(idx, (b, keep_num, nbhd)), axis=1)
    lp = jnp.take_along_axis(learned_prob, member_idx_d.reshape(b, -1, 1), axis=1).reshape(b, keep_num, nbhd, 1)

    def layernorm(x, g, bb, eps=1e-5):
        m = jnp.mean(x, axis=-1, keepdims=True)
        v = jnp.var(x, axis=-1, keepdims=True)
        return (x - m) / jnp.sqrt(v + eps) * g + bb

    wt = pre_table @ W1 + b1
    wt = layernorm(wt, ln1_g, ln1_b)
    wt = jax.nn.gelu(wt, approximate=False)
    weights = jnp.take(wt, pe_idx_d, axis=0)
    lp = lp * cluster_mask_d[..., None]
    weights = weights * lp
    gidx = (member_idx_d + (jnp.arange(b, dtype=jnp.int32) * n)[:, None, None]).reshape(-1)
    gathered = _sc_gather_rows(feat.reshape(b * n, c), gidx).reshape(b, keep_num, nbhd, c)
    feat_down = jnp.einsum('bnki,bnkc->bnic', weights, gathered).reshape(b, keep_num, -1)
    feat_down = _ln_proj(feat_down.reshape(b * keep_num, -1), norm_g, norm_b, W2, b2).reshape(b, keep_num, -1)
    return pos_down, feat_down


# speedup vs baseline: 18.2479x; 7.2589x over previous
import functools

import jax
import jax.numpy as jnp
from jax import lax
from jax.experimental import pallas as pl
from jax.experimental.pallas import tpu as pltpu
from jax.experimental.pallas import tpu_sc as plsc

DS_RATE = 0.25

_SC_INFO = plsc.get_sparse_core_info()
_NC, _NS = _SC_INFO.num_cores, _SC_INFO.num_subcores
_NW = _NC * _NS


def _sc_weighted_gather(feat_flat, member_glob, pe_flat, mask_flat, lp_flat,
                        wt, samp_glob, inner):
    BN, C = feat_flat.shape
    K = member_glob.shape[1]
    T = wt.shape[0]
    S = samp_glob.shape[0]
    tok_per_w = S // _NW
    assert S % _NW == 0 and tok_per_w % 16 == 0
    n_groups = tok_per_w // 16
    NCC = C // 16
    KC = K // 16
    OD = inner * C

    mesh = plsc.VectorSubcoreMesh(core_axis_name="c", subcore_axis_name="s")

    @functools.partial(
        pl.kernel,
        out_type=jax.ShapeDtypeStruct((S, OD), jnp.float32),
        mesh=mesh,
        compiler_params=pltpu.CompilerParams(needs_layout_passes=False,
                                             use_tc_tiling_on_sc=False),
        scratch_types=[
            pltpu.VMEM((tok_per_w,), jnp.int32),
            pltpu.VMEM((tok_per_w, K), jnp.int32),
            pltpu.VMEM((tok_per_w, K), jnp.int32),
            pltpu.VMEM((tok_per_w, K), jnp.float32),
            pltpu.VMEM((BN,), jnp.float32),
            pltpu.VMEM((T, inner), jnp.float32),
            pltpu.VMEM((K, C), jnp.float32),
            pltpu.VMEM((K, C), jnp.float32),
            pltpu.VMEM((8, OD), jnp.float32),
            pltpu.VMEM((8, OD), jnp.float32),
            pltpu.SemaphoreType.DMA,
            pltpu.SemaphoreType.DMA,
            pltpu.SemaphoreType.DMA,
            pltpu.SemaphoreType.DMA,
            pltpu.SemaphoreType.DMA,
        ],
    )
    def k(feat_hbm, member_hbm, pe_hbm, mask_hbm, lp_hbm, wt_hbm, samp_hbm,
          out_hbm, samp_v, gidx_sel, pe_sel, mask_sel, lp_v, wt_v,
          buf0, buf1, stage0, stage1, g0, g1, o0, o1, ssem):
        wid = lax.axis_index("s") * _NC + lax.axis_index("c")
        base = wid * tok_per_w
        bufs = (buf0, buf1)
        gsems = (g0, g1)
        stages = (stage0, stage1)
        osems = (o0, o1)

        pltpu.sync_copy(samp_hbm.at[pl.ds(base, tok_per_w)], samp_v)
        pltpu.sync_copy(lp_hbm, lp_v)
        pltpu.sync_copy(wt_hbm, wt_v)
        for lo in range(0, tok_per_w, 128):
            sl = pl.ds(lo, 128)
            pltpu.async_copy(member_hbm.at[samp_v.at[sl]], gidx_sel.at[sl], ssem)
            pltpu.async_copy(pe_hbm.at[samp_v.at[sl]], pe_sel.at[sl], ssem)
            pltpu.async_copy(mask_hbm.at[samp_v.at[sl]], mask_sel.at[sl], ssem)
            pltpu.make_async_copy(member_hbm.at[samp_v.at[sl]], gidx_sel.at[sl], ssem).wait()
            pltpu.make_async_copy(pe_hbm.at[samp_v.at[sl]], pe_sel.at[sl], ssem).wait()
            pltpu.make_async_copy(mask_hbm.at[samp_v.at[sl]], mask_sel.at[sl], ssem).wait()

        def start_gather(j, p):
            pltpu.async_copy(feat_hbm.at[gidx_sel.at[j]], bufs[p], gsems[p])

        def gather_wait(p):
            pltpu.make_async_copy(feat_hbm.at[gidx_sel.at[0]], bufs[p],
                                  gsems[p]).wait()

        start_gather(0, 0)

        def token_body(j, p, stage, u2):
            @pl.when(j < tok_per_w - 1)
            def _():
                start_gather(j + 1, 1 - p)

            gather_wait(p)
            rows = bufs[p]
            acc = [[jnp.zeros((16,), jnp.float32) for _ in range(NCC)]
                   for _ in range(inner)]
            for ch in range(KC):
                sl = pl.ds(ch * 16, 16)
                gm16 = gidx_sel[j, sl]
                pe16 = pe_sel[j, sl]
                lm = plsc.load_gather(lp_v, [gm16]) * mask_sel[j, sl]
                wv = [plsc.load_gather(wt_v, [pe16, jnp.full((16,), i, jnp.int32)]) * lm
                      for i in range(inner)]
                for kl in range(16):
                    kk = ch * 16 + kl
                    ws = [wv[i][kl] for i in range(inner)]
                    for cc in range(NCC):
                        rv = rows[kk, pl.ds(cc * 16, 16)]
                        for i in range(inner):
                            acc[i][cc] = acc[i][cc] + ws[i] * rv
            for i in range(inner):
                for cc in range(NCC):
                    stage[u2, pl.ds(i * C + cc * 16, 16)] = acc[i][cc]

        def flush_wait(h):
            pltpu.make_async_copy(stages[h], out_hbm.at[pl.ds(0, 8)],
                                  osems[h]).wait()

        def group_body(g, carry):
            for half in range(2):
                @pl.when(g >= 1)
                def _():
                    flush_wait(half)

                def pair_body(u, carry2):
                    for sub in range(2):
                        j = g * 16 + half * 8 + u * 2 + sub
                        token_body(j, sub, stages[half], u * 2 + sub)
                    return carry2

                lax.fori_loop(0, 4, pair_body, 0)
                pltpu.async_copy(
                    stages[half],
                    out_hbm.at[pl.ds(base + g * 16 + half * 8, 8)],
                    osems[half])
            return carry

        lax.fori_loop(0, n_groups, group_body, 0)
        flush_wait(0)
        flush_wait(1)

    return k(feat_flat, member_glob, pe_flat, mask_flat, lp_flat, wt,
             samp_glob)


def _tail_kernel(x_ref, g_ref, b_ref, w_ref, b2_ref, o_ref):
    x = x_ref[...]
    m = jnp.mean(x, axis=-1, keepdims=True)
    v = jnp.mean((x - m) ** 2, axis=-1, keepdims=True)
    xn = (x - m) * jax.lax.rsqrt(v + 1e-5) * g_ref[...] + b_ref[...]
    o_ref[...] = jnp.dot(xn, w_ref[...], preferred_element_type=jnp.float32) + b2_ref[...]


def _ln_proj(x, norm_g, norm_b, W2, b2):
    rows, d = x.shape
    out_d = W2.shape[1]
    blk = 1024
    return pl.pallas_call(
        _tail_kernel,
        grid=(rows // blk,),
        in_specs=[
            pl.BlockSpec((blk, d), lambda i: (i, 0)),
            pl.BlockSpec((1, d), lambda i: (0, 0)),
            pl.BlockSpec((1, d), lambda i: (0, 0)),
            pl.BlockSpec((d, out_d), lambda i: (0, 0)),
            pl.BlockSpec((1, out_d), lambda i: (0, 0)),
        ],
        out_specs=pl.BlockSpec((blk, out_d), lambda i: (i, 0)),
        out_shape=jax.ShapeDtypeStruct((rows, out_d), jnp.float32),
    )(x, norm_g.reshape(1, d), norm_b.reshape(1, d), W2, b2.reshape(1, out_d))


def kernel(pos, feat, member_idx, cluster_mask, learned_prob, stride, pe_idx,
           reserve_num, pre_table, W1, b1, ln1_g, ln1_b, norm_g, norm_b, W2, b2):
    b, n, c = feat.shape
    d = pos.shape[2]
    nbhd = member_idx.shape[-1]
    inner = W1.shape[1]
    keep_num = int(n * DS_RATE)

    final_prob = learned_prob.reshape(b, n)
    _, sample_idx = jax.lax.top_k(final_prob, keep_num)
    idx = sample_idx[:, :, None]
    pos_down = jnp.take_along_axis(pos, jnp.broadcast_to(idx, (b, keep_num, d)), axis=1)

    def layernorm(x, g, bb, eps=1e-5):
        m = jnp.mean(x, axis=-1, keepdims=True)
        v = jnp.var(x, axis=-1, keepdims=True)
        return (x - m) / jnp.sqrt(v + eps) * g + bb

    wt = pre_table @ W1 + b1
    wt = layernorm(wt, ln1_g, ln1_b)
    wt = jax.nn.gelu(wt, approximate=False)

    offs = (jnp.arange(b, dtype=jnp.int32) * n)[:, None]
    samp_glob = (sample_idx.astype(jnp.int32) + offs).reshape(-1)
    member_glob = (member_idx.astype(jnp.int32) + offs[:, :, None]).reshape(b * n, nbhd)

    feat_pre = _sc_weighted_gather(
        feat.reshape(b * n, c), member_glob,
        pe_idx.astype(jnp.int32).reshape(b * n, nbhd),
        cluster_mask.reshape(b * n, nbhd),
        learned_prob.reshape(b * n), wt, samp_glob, inner)

    feat_down = _ln_proj(feat_pre, norm_g, norm_b, W2, b2).reshape(b, keep_num, -1)
    return pos_down, feat_down


# speedup vs baseline: 29.9049x; 1.6388x over previous
import functools

import jax
import jax.numpy as jnp
from jax import lax
from jax.experimental import pallas as pl
from jax.experimental.pallas import tpu as pltpu
from jax.experimental.pallas import tpu_sc as plsc

DS_RATE = 0.25

_SC_INFO = plsc.get_sparse_core_info()
_NC, _NS = _SC_INFO.num_cores, _SC_INFO.num_subcores
_NW = _NC * _NS


def _sc_weighted_gather(feat_flat, member_glob, pe_flat, mask_flat, lp_flat,
                        wt, samp_glob, inner):
    BN, C = feat_flat.shape
    K = member_glob.shape[1]
    T = wt.shape[0]
    S = samp_glob.shape[0]
    tok_per_w = S // _NW
    assert S % _NW == 0 and tok_per_w % 16 == 0
    n_groups = tok_per_w // 16
    NCC = C // 16
    KC = K // 16
    OD = inner * C

    mesh = plsc.VectorSubcoreMesh(core_axis_name="c", subcore_axis_name="s")

    @functools.partial(
        pl.kernel,
        out_type=jax.ShapeDtypeStruct((S, OD), jnp.float32),
        mesh=mesh,
        compiler_params=pltpu.CompilerParams(needs_layout_passes=False,
                                             use_tc_tiling_on_sc=False),
        scratch_types=[
            pltpu.VMEM((tok_per_w,), jnp.int32),
            pltpu.VMEM((tok_per_w, K), jnp.int32),
            pltpu.VMEM((tok_per_w, K), jnp.int32),
            pltpu.VMEM((tok_per_w, K), jnp.float32),
            pltpu.VMEM((BN,), jnp.float32),
            pltpu.VMEM((T, inner), jnp.float32),
            pltpu.VMEM((K, C), jnp.float32),
            pltpu.VMEM((K, C), jnp.float32),
            pltpu.VMEM((K, C), jnp.float32),
            pltpu.VMEM((K, C), jnp.float32),
            pltpu.VMEM((8, OD), jnp.float32),
            pltpu.VMEM((8, OD), jnp.float32),
            pltpu.SemaphoreType.DMA,
            pltpu.SemaphoreType.DMA,
            pltpu.SemaphoreType.DMA,
            pltpu.SemaphoreType.DMA,
            pltpu.SemaphoreType.DMA,
            pltpu.SemaphoreType.DMA,
            pltpu.SemaphoreType.DMA,
        ],
    )
    def k(feat_hbm, member_hbm, pe_hbm, mask_hbm, lp_hbm, wt_hbm, samp_hbm,
          out_hbm, samp_v, gidx_sel, pe_sel, mask_sel, lp_v, wt_v,
          buf0, buf1, buf2, buf3, stage0, stage1, g0, g1, g2, g3, o0, o1, ssem):
        wid = lax.axis_index("s") * _NC + lax.axis_index("c")
        base = wid * tok_per_w
        bufs = (buf0, buf1, buf2, buf3)
        gsems = (g0, g1, g2, g3)
        stages = (stage0, stage1)
        osems = (o0, o1)
        DEPTH = 4

        pltpu.sync_copy(samp_hbm.at[pl.ds(base, tok_per_w)], samp_v)
        pltpu.sync_copy(lp_hbm, lp_v)
        pltpu.sync_copy(wt_hbm, wt_v)
        for lo in range(0, tok_per_w, 128):
            sl = pl.ds(lo, 128)
            pltpu.async_copy(member_hbm.at[samp_v.at[sl]], gidx_sel.at[sl], ssem)
            pltpu.async_copy(pe_hbm.at[samp_v.at[sl]], pe_sel.at[sl], ssem)
            pltpu.async_copy(mask_hbm.at[samp_v.at[sl]], mask_sel.at[sl], ssem)
            pltpu.make_async_copy(member_hbm.at[samp_v.at[sl]], gidx_sel.at[sl], ssem).wait()
            pltpu.make_async_copy(pe_hbm.at[samp_v.at[sl]], pe_sel.at[sl], ssem).wait()
            pltpu.make_async_copy(mask_hbm.at[samp_v.at[sl]], mask_sel.at[sl], ssem).wait()

        def start_gather(j, p):
            pltpu.async_copy(feat_hbm.at[gidx_sel.at[j]], bufs[p], gsems[p])

        def gather_wait(p):
            pltpu.make_async_copy(feat_hbm.at[gidx_sel.at[0]], bufs[p],
                                  gsems[p]).wait()

        for jj in range(DEPTH - 1):
            start_gather(jj, jj)

        def token_body(j, p, stage, u2):
            @pl.when(j < tok_per_w - (DEPTH - 1))
            def _():
                start_gather(j + DEPTH - 1, (p + DEPTH - 1) % DEPTH)

            gather_wait(p)
            rows = bufs[p]

            def chunk_body(ch, acc):
                sl = pl.ds(ch * 16, 16)
                gm16 = gidx_sel[j, sl]
                pe16 = pe_sel[j, sl]
                lm = plsc.load_gather(lp_v, [gm16]) * mask_sel[j, sl]
                wv = [plsc.load_gather(wt_v, [pe16, jnp.full((16,), i, jnp.int32)]) * lm
                      for i in range(inner)]
                acc = list(acc)
                for kl in range(16):
                    kk = ch * 16 + kl
                    ws = [wv[i][kl] for i in range(inner)]
                    for cc in range(NCC):
                        rv = rows[kk, pl.ds(cc * 16, 16)]
                        for i in range(inner):
                            acc[i * NCC + cc] = acc[i * NCC + cc] + ws[i] * rv
                return tuple(acc)

            acc0 = tuple(jnp.zeros((16,), jnp.float32)
                         for _ in range(inner * NCC))
            acc = lax.fori_loop(0, KC, chunk_body, acc0)
            for i in range(inner):
                for cc in range(NCC):
                    stage[u2, pl.ds(i * C + cc * 16, 16)] = acc[i * NCC + cc]

        def flush_wait(h):
            pltpu.make_async_copy(stages[h], out_hbm.at[pl.ds(0, 8)],
                                  osems[h]).wait()

        def group_body(g, carry):
            for half in range(2):
                @pl.when(g >= 1)
                def _():
                    flush_wait(half)

                def quad_body(u, carry2):
                    for sub in range(4):
                        j = g * 16 + half * 8 + u * 4 + sub
                        token_body(j, sub, stages[half], u * 4 + sub)
                    return carry2

                lax.fori_loop(0, 2, quad_body, 0)
                pltpu.async_copy(
                    stages[half],
                    out_hbm.at[pl.ds(base + g * 16 + half * 8, 8)],
                    osems[half])
            return carry

        lax.fori_loop(0, n_groups, group_body, 0)
        flush_wait(0)
        flush_wait(1)

    return k(feat_flat, member_glob, pe_flat, mask_flat, lp_flat, wt,
             samp_glob)


def _tail_kernel(x_ref, g_ref, b_ref, w_ref, b2_ref, o_ref):
    x = x_ref[...]
    m = jnp.mean(x, axis=-1, keepdims=True)
    v = jnp.mean((x - m) ** 2, axis=-1, keepdims=True)
    xn = (x - m) * jax.lax.rsqrt(v + 1e-5) * g_ref[...] + b_ref[...]
    o_ref[...] = jnp.dot(xn, w_ref[...], preferred_element_type=jnp.float32) + b2_ref[...]


def _ln_proj(x, norm_g, norm_b, W2, b2):
    rows, d = x.shape
    out_d = W2.shape[1]
    blk = 1024
    return pl.pallas_call(
        _tail_kernel,
        grid=(rows // blk,),
        in_specs=[
            pl.BlockSpec((blk, d), lambda i: (i, 0)),
            pl.BlockSpec((1, d), lambda i: (0, 0)),
            pl.BlockSpec((1, d), lambda i: (0, 0)),
            pl.BlockSpec((d, out_d), lambda i: (0, 0)),
            pl.BlockSpec((1, out_d), lambda i: (0, 0)),
        ],
        out_specs=pl.BlockSpec((blk, out_d), lambda i: (i, 0)),
        out_shape=jax.ShapeDtypeStruct((rows, out_d), jnp.float32),
    )(x, norm_g.reshape(1, d), norm_b.reshape(1, d), W2, b2.reshape(1, out_d))


def kernel(pos, feat, member_idx, cluster_mask, learned_prob, stride, pe_idx,
           reserve_num, pre_table, W1, b1, ln1_g, ln1_b, norm_g, norm_b, W2, b2):
    b, n, c = feat.shape
    d = pos.shape[2]
    nbhd = member_idx.shape[-1]
    inner = W1.shape[1]
    keep_num = int(n * DS_RATE)

    final_prob = learned_prob.reshape(b, n)
    _, sample_idx = jax.lax.top_k(final_prob, keep_num)
    idx = sample_idx[:, :, None]
    pos_down = jnp.take_along_axis(pos, jnp.broadcast_to(idx, (b, keep_num, d)), axis=1)

    def layernorm(x, g, bb, eps=1e-5):
        m = jnp.mean(x, axis=-1, keepdims=True)
        v = jnp.var(x, axis=-1, keepdims=True)
        return (x - m) / jnp.sqrt(v + eps) * g + bb

    wt = pre_table @ W1 + b1
    wt = layernorm(wt, ln1_g, ln1_b)
    wt = jax.nn.gelu(wt, approximate=False)

    offs = (jnp.arange(b, dtype=jnp.int32) * n)[:, None]
    samp_glob = (sample_idx.astype(jnp.int32) + offs).reshape(-1)
    member_glob = (member_idx.astype(jnp.int32) + offs[:, :, None]).reshape(b * n, nbhd)

    feat_pre = _sc_weighted_gather(
        feat.reshape(b * n, c), member_glob,
        pe_idx.astype(jnp.int32).reshape(b * n, nbhd),
        cluster_mask.reshape(b * n, nbhd),
        learned_prob.reshape(b * n), wt, samp_glob, inner)

    feat_down = _ln_proj(feat_pre, norm_g, norm_b, W2, b2).reshape(b, keep_num, -1)
    return pos_down, feat_down


# speedup vs baseline: 30.6500x; 1.0249x over previous
import functools

import jax
import jax.numpy as jnp
from jax import lax
from jax.experimental import pallas as pl
from jax.experimental.pallas import tpu as pltpu
from jax.experimental.pallas import tpu_sc as plsc

DS_RATE = 0.25

_SC_INFO = plsc.get_sparse_core_info()
_NC, _NS = _SC_INFO.num_cores, _SC_INFO.num_subcores
_NW = _NC * _NS


def _sc_weighted_gather(feat_flat, member_glob, pe_flat, mask_flat, lp_flat,
                        wt, samp_glob, inner, nbatch):
    BN, C = feat_flat.shape
    K = member_glob.shape[1]
    T = wt.shape[0]
    S = samp_glob.shape[0]
    tok_per_w = S // _NW
    assert S % _NW == 0 and tok_per_w % 16 == 0
    n_groups = tok_per_w // 16
    NCC = C // 16
    KC = K // 16
    OD = inner * C

    mesh = plsc.VectorSubcoreMesh(core_axis_name="c", subcore_axis_name="s")

    @functools.partial(
        pl.kernel,
        out_type=jax.ShapeDtypeStruct((S, OD), jnp.float32),
        mesh=mesh,
        compiler_params=pltpu.CompilerParams(needs_layout_passes=False,
                                             use_tc_tiling_on_sc=False),
        scratch_types=[
            pltpu.VMEM((tok_per_w,), jnp.int32),
            pltpu.VMEM((tok_per_w, K), jnp.int32),
            pltpu.VMEM((tok_per_w, K), jnp.int32),
            pltpu.VMEM((tok_per_w, K), jnp.float32),
            pltpu.VMEM((BN // nbatch,), jnp.float32),
            pltpu.VMEM((T, inner), jnp.float32),
            pltpu.VMEM((K, C), jnp.float32),
            pltpu.VMEM((K, C), jnp.float32),
            pltpu.VMEM((K, C), jnp.float32),
            pltpu.VMEM((K, C), jnp.float32),
            pltpu.VMEM((K, C), jnp.float32),
            pltpu.VMEM((K, C), jnp.float32),
            pltpu.VMEM((K, C), jnp.float32),
            pltpu.VMEM((K, C), jnp.float32),
            pltpu.VMEM((8, OD), jnp.float32),
            pltpu.VMEM((8, OD), jnp.float32),
            pltpu.SemaphoreType.DMA,
            pltpu.SemaphoreType.DMA,
            pltpu.SemaphoreType.DMA,
            pltpu.SemaphoreType.DMA,
            pltpu.SemaphoreType.DMA,
            pltpu.SemaphoreType.DMA,
            pltpu.SemaphoreType.DMA,
            pltpu.SemaphoreType.DMA,
            pltpu.SemaphoreType.DMA,
            pltpu.SemaphoreType.DMA,
            pltpu.SemaphoreType.DMA,
        ],
    )
    def k(feat_hbm, member_hbm, pe_hbm, mask_hbm, lp_hbm, wt_hbm, samp_hbm,
          out_hbm, samp_v, gidx_sel, pe_sel, mask_sel, lp_v, wt_v,
          buf0, buf1, buf2, buf3, buf4, buf5, buf6, buf7, stage0, stage1,
          g0, g1, g2, g3, g4, g5, g6, g7, o0, o1, ssem):
        wid = lax.axis_index("s") * _NC + lax.axis_index("c")
        base = wid * tok_per_w
        nloc = BN // nbatch
        batch = wid // (_NW // nbatch)
        boff = batch * nloc
        bufs = (buf0, buf1, buf2, buf3, buf4, buf5, buf6, buf7)
        gsems = (g0, g1, g2, g3, g4, g5, g6, g7)
        stages = (stage0, stage1)
        osems = (o0, o1)
        DEPTH = 8

        pltpu.sync_copy(samp_hbm.at[pl.ds(base, tok_per_w)], samp_v)
        pltpu.sync_copy(lp_hbm.at[pl.ds(boff, nloc)], lp_v)
        pltpu.sync_copy(wt_hbm, wt_v)
        for lo in range(0, tok_per_w, 128):
            sl = pl.ds(lo, 128)
            pltpu.async_copy(member_hbm.at[samp_v.at[sl]], gidx_sel.at[sl], ssem)
            pltpu.async_copy(pe_hbm.at[samp_v.at[sl]], pe_sel.at[sl], ssem)
            pltpu.async_copy(mask_hbm.at[samp_v.at[sl]], mask_sel.at[sl], ssem)
            pltpu.make_async_copy(member_hbm.at[samp_v.at[sl]], gidx_sel.at[sl], ssem).wait()
            pltpu.make_async_copy(pe_hbm.at[samp_v.at[sl]], pe_sel.at[sl], ssem).wait()
            pltpu.make_async_copy(mask_hbm.at[samp_v.at[sl]], mask_sel.at[sl], ssem).wait()

        def start_gather(j, p):
            pltpu.async_copy(feat_hbm.at[gidx_sel.at[j]], bufs[p], gsems[p])

        def gather_wait(p):
            pltpu.make_async_copy(feat_hbm.at[gidx_sel.at[0]], bufs[p],
                                  gsems[p]).wait()

        for jj in range(DEPTH - 1):
            start_gather(jj, jj)

        def token_body(j, p, stage, u2):
            @pl.when(j < tok_per_w - (DEPTH - 1))
            def _():
                start_gather(j + DEPTH - 1, (p + DEPTH - 1) % DEPTH)

            gather_wait(p)
            rows = bufs[p]

            def chunk_body(ch, acc):
                sl = pl.ds(ch * 16, 16)
                gm16 = gidx_sel[j, sl]
                pe16 = pe_sel[j, sl]
                lm = plsc.load_gather(lp_v, [gm16 - boff]) * mask_sel[j, sl]
                wv = [plsc.load_gather(wt_v, [pe16, jnp.full((16,), i, jnp.int32)]) * lm
                      for i in range(inner)]
                acc = list(acc)
                for kl in range(16):
                    kk = ch * 16 + kl
                    ws = [wv[i][kl] for i in range(inner)]
                    for cc in range(NCC):
                        rv = rows[kk, pl.ds(cc * 16, 16)]
                        for i in range(inner):
                            acc[i * NCC + cc] = acc[i * NCC + cc] + ws[i] * rv
                return tuple(acc)

            acc0 = tuple(jnp.zeros((16,), jnp.float32)
                         for _ in range(inner * NCC))
            acc = lax.fori_loop(0, KC, chunk_body, acc0)
            for i in range(inner):
                for cc in range(NCC):
                    stage[u2, pl.ds(i * C + cc * 16, 16)] = acc[i * NCC + cc]

        def flush_wait(h):
            pltpu.make_async_copy(stages[h], out_hbm.at[pl.ds(0, 8)],
                                  osems[h]).wait()

        def group_body(g, carry):
            for half in range(2):
                @pl.when(g >= 1)
                def _():
                    flush_wait(half)

                for sub in range(8):
                    j = g * 16 + half * 8 + sub
                    token_body(j, sub, stages[half], sub)
                pltpu.async_copy(
                    stages[half],
                    out_hbm.at[pl.ds(base + g * 16 + half * 8, 8)],
                    osems[half])
            return carry

        lax.fori_loop(0, n_groups, group_body, 0)
        flush_wait(0)
        flush_wait(1)

    return k(feat_flat, member_glob, pe_flat, mask_flat, lp_flat, wt,
             samp_glob)


def _tail_kernel(x_ref, g_ref, b_ref, w_ref, b2_ref, o_ref):
    x = x_ref[...]
    m = jnp.mean(x, axis=-1, keepdims=True)
    v = jnp.mean((x - m) ** 2, axis=-1, keepdims=True)
    xn = (x - m) * jax.lax.rsqrt(v + 1e-5) * g_ref[...] + b_ref[...]
    o_ref[...] = jnp.dot(xn, w_ref[...], preferred_element_type=jnp.float32) + b2_ref[...]


def _ln_proj(x, norm_g, norm_b, W2, b2):
    rows, d = x.shape
    out_d = W2.shape[1]
    blk = 1024
    return pl.pallas_call(
        _tail_kernel,
        grid=(rows // blk,),
        in_specs=[
            pl.BlockSpec((blk, d), lambda i: (i, 0)),
            pl.BlockSpec((1, d), lambda i: (0, 0)),
            pl.BlockSpec((1, d), lambda i: (0, 0)),
            pl.BlockSpec((d, out_d), lambda i: (0, 0)),
            pl.BlockSpec((1, out_d), lambda i: (0, 0)),
        ],
        out_specs=pl.BlockSpec((blk, out_d), lambda i: (i, 0)),
        out_shape=jax.ShapeDtypeStruct((rows, out_d), jnp.float32),
    )(x, norm_g.reshape(1, d), norm_b.reshape(1, d), W2, b2.reshape(1, out_d))


def kernel(pos, feat, member_idx, cluster_mask, learned_prob, stride, pe_idx,
           reserve_num, pre_table, W1, b1, ln1_g, ln1_b, norm_g, norm_b, W2, b2):
    b, n, c = feat.shape
    d = pos.shape[2]
    nbhd = member_idx.shape[-1]
    inner = W1.shape[1]
    keep_num = int(n * DS_RATE)

    final_prob = learned_prob.reshape(b, n)
    _, sample_idx = jax.lax.top_k(final_prob, keep_num)
    idx = sample_idx[:, :, None]
    pos_down = jnp.take_along_axis(pos, jnp.broadcast_to(idx, (b, keep_num, d)), axis=1)

    def layernorm(x, g, bb, eps=1e-5):
        m = jnp.mean(x, axis=-1, keepdims=True)
        v = jnp.var(x, axis=-1, keepdims=True)
        return (x - m) / jnp.sqrt(v + eps) * g + bb

    wt = pre_table @ W1 + b1
    wt = layernorm(wt, ln1_g, ln1_b)
    wt = jax.nn.gelu(wt, approximate=False)

    offs = (jnp.arange(b, dtype=jnp.int32) * n)[:, None]
    samp_glob = (sample_idx.astype(jnp.int32) + offs).reshape(-1)
    member_glob = (member_idx.astype(jnp.int32) + offs[:, :, None]).reshape(b * n, nbhd)

    feat_pre = _sc_weighted_gather(
        feat.reshape(b * n, c), member_glob,
        pe_idx.astype(jnp.int32).reshape(b * n, nbhd),
        cluster_mask.reshape(b * n, nbhd),
        learned_prob.reshape(b * n), wt, samp_glob, inner, b)

    feat_down = _ln_proj(feat_pre, norm_g, norm_b, W2, b2).reshape(b, keep_num, -1)
    return pos_down, feat_down


# speedup vs baseline: 30.7635x; 1.0037x over previous
import functools

import jax
import jax.numpy as jnp
from jax import lax
from jax.experimental import pallas as pl
from jax.experimental.pallas import tpu as pltpu
from jax.experimental.pallas import tpu_sc as plsc

DS_RATE = 0.25

_SC_INFO = plsc.get_sparse_core_info()
_NC, _NS = _SC_INFO.num_cores, _SC_INFO.num_subcores
_NW = _NC * _NS


def _sc_weighted_gather(feat_flat, member_glob, pe_flat, mask_flat, lp_flat,
                        wt, samp_glob, inner, nbatch):
    BN, C = feat_flat.shape
    K = member_glob.shape[1]
    T = wt.shape[0]
    S = samp_glob.shape[0]
    tok_per_w = S // _NW
    assert S % _NW == 0 and tok_per_w % 16 == 0
    n_groups = tok_per_w // 16
    NCC = C // 16
    KC = K // 16
    OD = inner * C

    mesh = plsc.VectorSubcoreMesh(core_axis_name="c", subcore_axis_name="s")

    @functools.partial(
        pl.kernel,
        out_type=jax.ShapeDtypeStruct((S, OD), jnp.float32),
        mesh=mesh,
        compiler_params=pltpu.CompilerParams(needs_layout_passes=False,
                                             use_tc_tiling_on_sc=False),
        scratch_types=[
            pltpu.VMEM((tok_per_w,), jnp.int32),
            pltpu.VMEM((tok_per_w, K), jnp.int32),
            pltpu.VMEM((tok_per_w, K), jnp.int32),
            pltpu.VMEM((tok_per_w, K), jnp.float32),
            pltpu.VMEM((BN // nbatch,), jnp.float32),
            pltpu.VMEM((T, inner), jnp.float32),
            pltpu.VMEM((K, C), jnp.float32),
            pltpu.VMEM((K, C), jnp.float32),
            pltpu.VMEM((K, C), jnp.float32),
            pltpu.VMEM((K, C), jnp.float32),
            pltpu.VMEM((K, C), jnp.float32),
            pltpu.VMEM((K, C), jnp.float32),
            pltpu.VMEM((K, C), jnp.float32),
            pltpu.VMEM((K, C), jnp.float32),
            pltpu.VMEM((8, OD), jnp.float32),
            pltpu.VMEM((8, OD), jnp.float32),
            pltpu.SemaphoreType.DMA,
            pltpu.SemaphoreType.DMA,
            pltpu.SemaphoreType.DMA,
            pltpu.SemaphoreType.DMA,
            pltpu.SemaphoreType.DMA,
            pltpu.SemaphoreType.DMA,
            pltpu.SemaphoreType.DMA,
            pltpu.SemaphoreType.DMA,
            pltpu.SemaphoreType.DMA,
            pltpu.SemaphoreType.DMA,
            pltpu.SemaphoreType.DMA,
        ],
    )
    def k(feat_hbm, member_hbm, pe_hbm, mask_hbm, lp_hbm, wt_hbm, samp_hbm,
          out_hbm, samp_v, gidx_sel, pe_sel, mask_sel, lp_v, wt_v,
          buf0, buf1, buf2, buf3, buf4, buf5, buf6, buf7, stage0, stage1,
          g0, g1, g2, g3, g4, g5, g6, g7, o0, o1, ssem):
        wid = lax.axis_index("s") * _NC + lax.axis_index("c")
        base = wid * tok_per_w
        nloc = BN // nbatch
        batch = wid // (_NW // nbatch)
        boff = batch * nloc
        bufs = (buf0, buf1, buf2, buf3, buf4, buf5, buf6, buf7)
        gsems = (g0, g1, g2, g3, g4, g5, g6, g7)
        stages = (stage0, stage1)
        osems = (o0, o1)
        DEPTH = 8

        pltpu.sync_copy(samp_hbm.at[pl.ds(base, tok_per_w)], samp_v)
        pltpu.sync_copy(lp_hbm.at[pl.ds(boff, nloc)], lp_v)
        pltpu.sync_copy(wt_hbm, wt_v)
        for lo in range(0, tok_per_w, 128):
            sl = pl.ds(lo, 128)
            pltpu.async_copy(member_hbm.at[samp_v.at[sl]], gidx_sel.at[sl], ssem)
            pltpu.async_copy(pe_hbm.at[samp_v.at[sl]], pe_sel.at[sl], ssem)
            pltpu.async_copy(mask_hbm.at[samp_v.at[sl]], mask_sel.at[sl], ssem)
            pltpu.make_async_copy(member_hbm.at[samp_v.at[sl]], gidx_sel.at[sl], ssem).wait()
            pltpu.make_async_copy(pe_hbm.at[samp_v.at[sl]], pe_sel.at[sl], ssem).wait()
            pltpu.make_async_copy(mask_hbm.at[samp_v.at[sl]], mask_sel.at[sl], ssem).wait()

        def start_gather(j, p):
            pltpu.async_copy(feat_hbm.at[gidx_sel.at[j]], bufs[p], gsems[p])

        def gather_wait(p):
            pltpu.make_async_copy(feat_hbm.at[gidx_sel.at[0]], bufs[p],
                                  gsems[p]).wait()

        for jj in range(DEPTH - 1):
            start_gather(jj, jj)

        def token_body(j, p, stage, u2):
            @pl.when(j < tok_per_w - (DEPTH - 1))
            def _():
                start_gather(j + DEPTH - 1, (p + DEPTH - 1) % DEPTH)

            gather_wait(p)
            rows = bufs[p]

            def chunk_body(ch, acc):
                sl = pl.ds(ch * 16, 16)
                gm16 = gidx_sel[j, sl]
                pe16 = pe_sel[j, sl]
                lm = plsc.load_gather(lp_v, [gm16 - boff]) * mask_sel[j, sl]
                wv = [plsc.load_gather(wt_v, [pe16, jnp.full((16,), i, jnp.int32)]) * lm
                      for i in range(inner)]
                acc = list(acc)
                for kl in range(16):
                    kk = ch * 16 + kl
                    ws = [wv[i][kl] for i in range(inner)]
                    for cc in range(NCC):
                        rv = rows[kk, pl.ds(cc * 16, 16)]
                        for i in range(inner):
                            acc[i * NCC + cc] = acc[i * NCC + cc] + ws[i] * rv
                return tuple(acc)

            acc0 = tuple(jnp.zeros((16,), jnp.float32)
                         for _ in range(inner * NCC))
            acc = lax.fori_loop(0, KC, chunk_body, acc0)
            for i in range(inner):
                for cc in range(NCC):
                    stage[u2, pl.ds(i * C + cc * 16, 16)] = acc[i * NCC + cc]

        def flush_wait(h):
            pltpu.make_async_copy(stages[h], out_hbm.at[pl.ds(0, 8)],
                                  osems[h]).wait()

        def group_body(g, carry):
            for half in range(2):
                @pl.when(g >= 1)
                def _():
                    flush_wait(half)

                for sub in range(8):
                    j = g * 16 + half * 8 + sub
                    token_body(j, sub, stages[half], sub)
                pltpu.async_copy(
                    stages[half],
                    out_hbm.at[pl.ds(base + g * 16 + half * 8, 8)],
                    osems[half])
            return carry

        lax.fori_loop(0, n_groups, group_body, 0)
        flush_wait(0)
        flush_wait(1)

    return k(feat_flat, member_glob, pe_flat, mask_flat, lp_flat, wt,
             samp_glob)


def _wt_kernel(p_ref, w1_ref, b1_ref, g_ref, b_ref, m_ref, o_ref, *, inner):
    x = jnp.dot(p_ref[...], w1_ref[...], preferred_element_type=jnp.float32)
    x = x + b1_ref[...]
    msk = m_ref[...]
    mu = jnp.sum(x * msk, axis=-1, keepdims=True) * (1.0 / inner)
    xc = (x - mu) * msk
    var = jnp.sum(xc * xc, axis=-1, keepdims=True) * (1.0 / inner)
    xn = xc * jax.lax.rsqrt(var + 1e-5) * g_ref[...] + b_ref[...]
    o_ref[...] = xn * 0.5 * (1.0 + jax.lax.erf(xn * (2.0 ** -0.5)))


def _weight_net(pre_table, W1, b1, ln1_g, ln1_b):
    t, five = pre_table.shape
    inner = W1.shape[1]
    tp = (t + 127) // 128 * 128
    pre_pad = jnp.zeros((tp, 128), jnp.float32).at[:t, :five].set(pre_table)
    w1_pad = jnp.zeros((128, 128), jnp.float32).at[:five, :inner].set(W1)
    b1_pad = jnp.zeros((1, 128), jnp.float32).at[0, :inner].set(b1)
    g_pad = jnp.zeros((1, 128), jnp.float32).at[0, :inner].set(ln1_g)
    b_pad = jnp.zeros((1, 128), jnp.float32).at[0, :inner].set(ln1_b)
    msk = jnp.zeros((1, 128), jnp.float32).at[0, :inner].set(1.0)
    out = pl.pallas_call(
        functools.partial(_wt_kernel, inner=inner),
        out_shape=jax.ShapeDtypeStruct((tp, 128), jnp.float32),
    )(pre_pad, w1_pad, b1_pad, g_pad, b_pad, msk)
    return out[:t, :inner]


def _tail_kernel(x_ref, g_ref, b_ref, w_ref, b2_ref, o_ref):
    x = x_ref[...]
    m = jnp.mean(x, axis=-1, keepdims=True)
    v = jnp.mean((x - m) ** 2, axis=-1, keepdims=True)
    xn = (x - m) * jax.lax.rsqrt(v + 1e-5) * g_ref[...] + b_ref[...]
    o_ref[...] = jnp.dot(xn, w_ref[...], preferred_element_type=jnp.float32) + b2_ref[...]


def _ln_proj(x, norm_g, norm_b, W2, b2):
    rows, d = x.shape
    out_d = W2.shape[1]
    blk = 1024
    return pl.pallas_call(
        _tail_kernel,
        grid=(rows // blk,),
        in_specs=[
            pl.BlockSpec((blk, d), lambda i: (i, 0)),
            pl.BlockSpec((1, d), lambda i: (0, 0)),
            pl.BlockSpec((1, d), lambda i: (0, 0)),
            pl.BlockSpec((d, out_d), lambda i: (0, 0)),
            pl.BlockSpec((1, out_d), lambda i: (0, 0)),
        ],
        out_specs=pl.BlockSpec((blk, out_d), lambda i: (i, 0)),
        out_shape=jax.ShapeDtypeStruct((rows, out_d), jnp.float32),
    )(x, norm_g.reshape(1, d), norm_b.reshape(1, d), W2, b2.reshape(1, out_d))


def kernel(pos, feat, member_idx, cluster_mask, learned_prob, stride, pe_idx,
           reserve_num, pre_table, W1, b1, ln1_g, ln1_b, norm_g, norm_b, W2, b2):
    b, n, c = feat.shape
    d = pos.shape[2]
    nbhd = member_idx.shape[-1]
    inner = W1.shape[1]
    keep_num = int(n * DS_RATE)

    final_prob = learned_prob.reshape(b, n)
    _, sample_idx = jax.lax.top_k(final_prob, keep_num)
    idx = sample_idx[:, :, None]
    pos_down = jnp.take_along_axis(pos, jnp.broadcast_to(idx, (b, keep_num, d)), axis=1)

    wt = _weight_net(pre_table, W1, b1, ln1_g, ln1_b)

    offs = (jnp.arange(b, dtype=jnp.int32) * n)[:, None]
    samp_glob = (sample_idx.astype(jnp.int32) + offs).reshape(-1)
    member_glob = (member_idx.astype(jnp.int32) + offs[:, :, None]).reshape(b * n, nbhd)

    feat_pre = _sc_weighted_gather(
        feat.reshape(b * n, c), member_glob,
        pe_idx.astype(jnp.int32).reshape(b * n, nbhd),
        cluster_mask.reshape(b * n, nbhd),
        learned_prob.reshape(b * n), wt, samp_glob, inner, b)

    feat_down = _ln_proj(feat_pre, norm_g, norm_b, W2, b2).reshape(b, keep_num, -1)
    return pos_down, feat_down
